# Initial kernel scaffold; baseline (speedup 1.0000x reference)
import jax
import jax.numpy as jnp
from jax.experimental import pallas as pl


def kernel(image_nodes, obj_nodes, pred_emb, sem_node_emb, sem_rel_emb, sem_similarity, rel_ind, nodes_mask, edges_mask, sem_node_idx, sem_mask):
    raise NotImplementedError("write your pallas kernel here")


# speedup vs baseline: 1.8149x; 1.8149x over previous
import jax
import jax.numpy as jnp
from jax.experimental import pallas as pl

_B, _N, _E, _S, _D = 8, 512, 2048, 128, 512
HI = jax.lax.Precision.HIGHEST


def _dot_t(a, m):
    return jax.lax.dot_general(a, m, (((1,), (1,)), ((), ())), precision=HI)


def _body(image_ref, obj_ref, pred_ref, sne_ref, sre_ref,
          ssim_ref, ri_flat_ref, nm_row_ref, em_row_ref, nm_col_ref,
          em_col_ref, sni_ref, smask_b_ref, smask_all_ref,
          img_out, nodes_out, edges_out, ri_out, nmask_out, emask_out, sim_out):
    S = _S
    f32 = jnp.float32

    smask_all = smask_all_ref[...].astype(f32)
    max_sv = jnp.max(jnp.sum(smask_all, axis=1))
    padf = f32(_N - 1) + max_sv

    v = smask_b_ref[0].astype(f32)
    s_v = jnp.sum(v)
    tri = (jax.lax.broadcasted_iota(jnp.int32, (S, S), 0)
           <= jax.lax.broadcasted_iota(jnp.int32, (S, S), 1)).astype(f32)
    c1 = jnp.dot(v, tri, precision=HI)
    c0 = jnp.dot(1.0 - v, tri, precision=HI)
    pos = jnp.where(v > 0.5, c1 - 1.0, s_v + c0 - 1.0)
    k_iota = jax.lax.broadcasted_iota(jnp.int32, (S, S), 0).astype(f32)
    P = (pos == k_iota).astype(f32)
    k_col = jax.lax.broadcasted_iota(jnp.int32, (S, 1), 0).astype(f32)
    validc = (k_col < s_v).astype(f32)
    k_row = jax.lax.broadcasted_iota(jnp.int32, (1, S), 1).astype(f32)
    validr = (k_row < s_v).astype(f32)

    def norm_compact(x):
        ss = jnp.sum(x * x, axis=1, keepdims=True)
        xn = x * jax.lax.rsqrt(ss)
        return jnp.dot(P, xn, precision=HI) * validc

    nodes_out[0, :_N, :] = obj_ref[0] * nm_col_ref[0]
    nodes_out[0, _N:, :] = norm_compact(sne_ref[0])
    edges_out[0, :_E, :] = pred_ref[0] * em_col_ref[0]
    edges_out[0, _E:, :] = norm_compact(sre_ref[0])

    img_out[0, :_N, :] = image_ref[0]
    img_out[0, _N:, :] = jnp.zeros((S, _D), f32)

    sni_c = _dot_t(sni_ref[0].astype(f32), P)
    first = jnp.where(validr > 0.5, f32(_N) + k_row, padf)
    second = jnp.where(validr > 0.5, sni_c, padf)
    i2 = jax.lax.broadcasted_iota(jnp.int32, (2 * S, S), 0).astype(f32)
    j2 = jax.lax.broadcasted_iota(jnp.int32, (2 * S, S), 1).astype(f32)
    A = (i2 == 2.0 * j2).astype(f32)
    Bm = (i2 == 2.0 * j2 + 1.0).astype(f32)
    tail_flat = _dot_t(first, A) + _dot_t(second, Bm)
    ri_out[0, 0, : 2 * _E] = ri_flat_ref[0, 0, :]
    ri_out[0, 0, 2 * _E:] = tail_flat[0].astype(jnp.int32)

    nmask_out[0, 0, :_N] = nm_row_ref[0, 0, :].astype(jnp.int32)
    nmask_out[0, 0, _N:] = validr[0].astype(jnp.int32)
    emask_out[0, 0, :_E] = em_row_ref[0, 0, :].astype(jnp.int32)
    emask_out[0, 0, _E:] = validr[0].astype(jnp.int32)
    ssim_c = _dot_t(ssim_ref[0], P)
    sim_out[0, 0, :_E] = em_row_ref[0, 0, :].astype(f32)
    sim_out[0, 0, _E:] = (ssim_c * validr)[0]


@jax.jit
def kernel(image_nodes, obj_nodes, pred_emb, sem_node_emb, sem_rel_emb,
           sem_similarity, rel_ind, nodes_mask, edges_mask, sem_node_idx,
           sem_mask):
    B, N, D = obj_nodes.shape
    E = pred_emb.shape[1]
    S = sem_node_emb.shape[1]
    nm_f = nodes_mask.astype(jnp.float32)
    em_f = edges_mask.astype(jnp.float32)
    smask_i = sem_mask.astype(jnp.int32)

    row3 = lambda X: pl.BlockSpec((1, 1, X), lambda b: (b, 0, 0))
    col3 = lambda R: pl.BlockSpec((1, R, 1), lambda b: (b, 0, 0))
    big = lambda R: pl.BlockSpec((1, R, D), lambda b: (b, 0, 0))
    mat = lambda X: pl.BlockSpec((1, S, X), lambda b: (b, 0, 0))

    outs = pl.pallas_call(
        _body,
        grid=(B,),
        in_specs=[
            big(N), big(N), big(E), big(S), big(S),
            pl.BlockSpec((1, 1, S), lambda b: (b, 0, 0)),
            row3(2 * E),
            row3(N), row3(E),
            col3(N), col3(E),
            pl.BlockSpec((1, 1, S), lambda b: (b, 0, 0)),
            pl.BlockSpec((1, 1, S), lambda b: (b, 0, 0)),
            pl.BlockSpec((B, S), lambda b: (0, 0)),
        ],
        out_specs=[
            big(N + S), big(N + S), big(E + S),
            row3((E + S) * 2), row3(N + S), row3(E + S), row3(E + S),
        ],
        out_shape=[
            jax.ShapeDtypeStruct((B, N + S, D), jnp.float32),
            jax.ShapeDtypeStruct((B, N + S, D), jnp.float32),
            jax.ShapeDtypeStruct((B, E + S, D), jnp.float32),
            jax.ShapeDtypeStruct((B, 1, (E + S) * 2), jnp.int32),
            jax.ShapeDtypeStruct((B, 1, N + S), jnp.int32),
            jax.ShapeDtypeStruct((B, 1, E + S), jnp.int32),
            jax.ShapeDtypeStruct((B, 1, E + S), jnp.float32),
        ],
    )(image_nodes, obj_nodes, pred_emb, sem_node_emb, sem_rel_emb,
      sem_similarity.reshape(B, 1, S), rel_ind.reshape(B, 1, E * 2),
      nm_f.reshape(B, 1, N), em_f.reshape(B, 1, E),
      nm_f.reshape(B, N, 1), em_f.reshape(B, E, 1),
      sem_node_idx.reshape(B, 1, S), smask_i.reshape(B, 1, S), smask_i)

    ext_image, ext_nodes, ext_edges, ri_o, nmask_o, emask_o, sim = outs
    return (ext_image, ext_nodes, ext_edges,
            ri_o.reshape(B, E + S, 2),
            nmask_o.reshape(B, N + S).astype(bool),
            emask_o.reshape(B, E + S).astype(bool),
            sim.reshape(B, E + S))
